# Initial kernel scaffold; baseline (speedup 1.0000x reference)
import jax
import jax.numpy as jnp
from jax.experimental import pallas as pl


def kernel(patch_features, edges, W1, b1, W2, b2, Wg, a_src, a_dst):
    raise NotImplementedError("write your pallas kernel here")


# speedup vs baseline: 78.1369x; 78.1369x over previous
import functools

import jax
import jax.numpy as jnp
from jax import lax
from jax.experimental import pallas as pl
from jax.experimental.pallas import tpu as pltpu
from jax.experimental.pallas import tpu_sc as plsc

B = 2
N = 10000
E = 320000
IN = 128
HID = 128
OUT = 128
HEADS = 2
DH = 64
F = HEADS * DH

NC = 2
NS = 16

CHUNK = 128
TOTAL_CHUNKS = E // CHUNK
BASE_CH = TOTAL_CHUNKS // NS
EXTRA_CH = TOTAL_CHUNKS - BASE_CH * NS
NPT = 624
NTAIL = N - NPT * NS
NP2 = 10240
BLK = 1000


def _tc_body(pf_ref, W1_ref, b1_ref, W2_ref, b2_ref, Wg_ref, A_ref,
             x_ref, g_ref, esed_ref):
    pf = pf_ref[0]
    h = jax.nn.gelu(jnp.dot(pf, W1_ref[...], preferred_element_type=jnp.float32)
                    + b1_ref[...])
    x = jnp.dot(h, W2_ref[...], preferred_element_type=jnp.float32) + b2_ref[...]
    g = jnp.dot(x, Wg_ref[...], preferred_element_type=jnp.float32)
    esed = jnp.dot(g, A_ref[...], preferred_element_type=jnp.float32)
    x_ref[0] = x
    g_ref[...] = g
    esed_ref[0] = esed


def _tc2_body(x_ref, a0_ref, a1_ref, out_ref):
    a = jnp.concatenate([a0_ref[0, 0], a1_ref[0, 0]], axis=-1)
    e = jnp.where(a > 0, a, jnp.exp(jnp.minimum(a, 0.0)) - 1.0)
    out_ref[0] = x_ref[0] + e


def _sc_body(edges, esed_h, ghead, zagg, zden, aggout,
             esed_t, den_t, idx_s, idx_d, idx_g, pay0, pay1, rows,
             den_sh, agg_sh, sem):
    c = lax.axis_index("c")
    s = lax.axis_index("s")
    nb = s * NPT

    pltpu.sync_copy(zden, den_sh.at[pl.ds(s * (2 * NP2 // NS), 2 * NP2 // NS)])
    pltpu.sync_copy(esed_h.at[c], esed_t)
    plsc.subcore_barrier()

    start = s * BASE_CH + jnp.minimum(s, EXTRA_CH)
    nch = BASE_CH + jnp.where(s < EXTRA_CH, 1, 0)

    def load_idx(ch):
        eb = ch * CHUNK
        pltpu.sync_copy(edges.at[0].at[pl.ds(eb, CHUNK)], idx_s)
        pltpu.sync_copy(edges.at[1].at[pl.ds(eb, CHUNK)], idx_d)

    def logits(sv, dv, h):
        es_v = plsc.load_gather(esed_t, [sv * 4 + h])
        ed_v = plsc.load_gather(esed_t, [dv * 4 + HEADS + h])
        e = es_v + ed_v
        return jnp.where(e > 0, e, 0.2 * e)

    def p1(i, _):
        load_idx(start + i)
        for j in range(CHUNK // 16):
            sl = pl.ds(j * 16, 16)
            sv = idx_s[sl]
            dv = idx_d[sl]
            idx_g[sl] = dv + NP2
            pay0[sl] = jnp.exp(logits(sv, dv, 0))
            pay1[sl] = jnp.exp(logits(sv, dv, 1))
        pltpu.sync_copy(pay0, den_sh.at[idx_d], add=True)
        pltpu.sync_copy(pay1, den_sh.at[idx_g], add=True)
        return 0

    lax.fori_loop(0, nch, p1, 0)
    plsc.subcore_barrier()

    pltpu.sync_copy(den_sh, den_t)

    for h in range(HEADS):
        pltpu.sync_copy(zagg, agg_sh.at[pl.ds(nb, NPT)])

        @pl.when(s == NS - 1)
        def _():
            pltpu.sync_copy(zagg.at[pl.ds(0, NTAIL)],
                            agg_sh.at[pl.ds(NPT * NS, NTAIL)])

        plsc.subcore_barrier()

        def p2(i, _):
            load_idx(start + i)
            for j in range(CHUNK // 16):
                sl = pl.ds(j * 16, 16)
                idx_g[sl] = (idx_s[sl] + c * N) * 2 + h
            cp = pltpu.async_copy(ghead.at[idx_g], rows, sem)
            for j in range(CHUNK // 16):
                sl = pl.ds(j * 16, 16)
                sv = idx_s[sl]
                dv = idx_d[sl]
                dh = plsc.load_gather(den_t, [dv + h * NP2])
                pay0[sl] = jnp.exp(logits(sv, dv, h)) / (dh + 1e-16)
            cp.wait()

            def scale(jj, _):
                av = plsc.load_gather(pay0, [jnp.full((16,), jj, jnp.int32)])
                for k in range(4):
                    rows[jj, pl.ds(k * 16, 16)] = rows[jj, pl.ds(k * 16, 16)] * av
                return 0

            lax.fori_loop(0, CHUNK, scale, 0)
            pltpu.sync_copy(rows, agg_sh.at[idx_d], add=True)
            return 0

        lax.fori_loop(0, nch, p2, 0)
        plsc.subcore_barrier()

        pltpu.sync_copy(agg_sh.at[pl.ds(nb, NPT)],
                        aggout.at[h].at[pl.ds(c * N + nb, NPT)])

        @pl.when(s == NS - 1)
        def _():
            pltpu.sync_copy(agg_sh.at[pl.ds(NPT * NS, NTAIL)],
                            aggout.at[h].at[pl.ds(c * N + NPT * NS, NTAIL)])


_sc_kernel = functools.partial(
    pl.kernel,
    out_type=jax.ShapeDtypeStruct((HEADS, B * N, DH), jnp.float32),
    mesh=plsc.VectorSubcoreMesh(core_axis_name="c", subcore_axis_name="s",
                                num_cores=NC, num_subcores=NS),
    compiler_params=pltpu.CompilerParams(needs_layout_passes=False,
                                         use_tc_tiling_on_sc=False),
    scratch_types=[
        pltpu.VMEM((N * 2 * HEADS,), jnp.float32),
        pltpu.VMEM((2 * NP2,), jnp.float32),
        pltpu.VMEM((CHUNK,), jnp.int32),
        pltpu.VMEM((CHUNK,), jnp.int32),
        pltpu.VMEM((CHUNK,), jnp.int32),
        pltpu.VMEM((CHUNK,), jnp.float32),
        pltpu.VMEM((CHUNK,), jnp.float32),
        pltpu.VMEM((CHUNK, DH), jnp.float32),
        pltpu.VMEM_SHARED((2 * NP2,), jnp.float32),
        pltpu.VMEM_SHARED((N, DH), jnp.float32),
        pltpu.SemaphoreType.DMA,
    ],
)(_sc_body)


def kernel(patch_features, edges, W1, b1, W2, b2, Wg, a_src, a_dst):
    z = jnp.zeros((DH,), jnp.float32)
    A = jnp.stack([
        jnp.concatenate([a_src[0], z]),
        jnp.concatenate([z, a_src[1]]),
        jnp.concatenate([a_dst[0], z]),
        jnp.concatenate([z, a_dst[1]]),
    ], axis=1)

    x, gflat, esed = pl.pallas_call(
        _tc_body,
        grid=(B, N // BLK),
        in_specs=[
            pl.BlockSpec((1, BLK, IN), lambda b, i: (b, i, 0)),
            pl.BlockSpec((IN, HID), lambda b, i: (0, 0)),
            pl.BlockSpec((1, HID), lambda b, i: (0, 0)),
            pl.BlockSpec((HID, OUT), lambda b, i: (0, 0)),
            pl.BlockSpec((1, OUT), lambda b, i: (0, 0)),
            pl.BlockSpec((OUT, F), lambda b, i: (0, 0)),
            pl.BlockSpec((F, 2 * HEADS), lambda b, i: (0, 0)),
        ],
        out_specs=[
            pl.BlockSpec((1, BLK, OUT), lambda b, i: (b, i, 0)),
            pl.BlockSpec((BLK, F), lambda b, i: (b * (N // BLK) + i, 0)),
            pl.BlockSpec((1, BLK, 2 * HEADS), lambda b, i: (b, i, 0)),
        ],
        out_shape=[
            jax.ShapeDtypeStruct((B, N, OUT), jnp.float32),
            jax.ShapeDtypeStruct((B * N, F), jnp.float32),
            jax.ShapeDtypeStruct((B, N, 2 * HEADS), jnp.float32),
        ],
    )(patch_features, W1, b1.reshape(1, HID), W2, b2.reshape(1, OUT), Wg, A)

    ghead = gflat.reshape(B * N * HEADS, DH)
    zagg = jnp.zeros((NPT, DH), jnp.float32)
    zden = jnp.zeros((2 * NP2 // NS,), jnp.float32)
    agg = _sc_kernel(edges, esed.reshape(B, N * 2 * HEADS), ghead, zagg, zden)

    return pl.pallas_call(
        _tc2_body,
        grid=(B, N // BLK),
        in_specs=[
            pl.BlockSpec((1, BLK, OUT), lambda b, i: (b, i, 0)),
            pl.BlockSpec((1, 1, BLK, DH), lambda b, i: (0, b, i, 0)),
            pl.BlockSpec((1, 1, BLK, DH), lambda b, i: (1, b, i, 0)),
        ],
        out_specs=pl.BlockSpec((1, BLK, OUT), lambda b, i: (b, i, 0)),
        out_shape=jax.ShapeDtypeStruct((B, N, OUT), jnp.float32),
    )(x, agg.reshape(HEADS, B, N, DH), agg.reshape(HEADS, B, N, DH))


# speedup vs baseline: 140.5603x; 1.7989x over previous
import functools

import jax
import jax.numpy as jnp
from jax import lax
from jax.experimental import pallas as pl
from jax.experimental.pallas import tpu as pltpu
from jax.experimental.pallas import tpu_sc as plsc

B = 2
N = 10000
E = 320000
IN = 128
HID = 128
OUT = 128
HEADS = 2
DH = 64
F = HEADS * DH

NC = 2
NS = 16

CHUNK = 128
NCHT = 158
EPT = NCHT * CHUNK
E_P = EPT * NS
NPT = 624
NTAIL = N - NPT * NS
NP2 = 10240
BLK = 1000


def _tc_body(pf_ref, W1_ref, b1_ref, W2_ref, b2_ref, Wg_ref, A_ref,
             x_ref, g_ref, esed_ref):
    pf = pf_ref[0]
    h = jax.nn.gelu(jnp.dot(pf, W1_ref[...], preferred_element_type=jnp.float32)
                    + b1_ref[...])
    x = jnp.dot(h, W2_ref[...], preferred_element_type=jnp.float32) + b2_ref[...]
    g = jnp.dot(x, Wg_ref[...], preferred_element_type=jnp.float32)
    esed = jnp.dot(g, A_ref[...], preferred_element_type=jnp.float32)
    x_ref[0] = x
    g_ref[...] = g
    esed_ref[0] = esed


def _tc2_body(x_ref, a0_ref, a1_ref, out_ref):
    a = jnp.concatenate([a0_ref[0, 0], a1_ref[0, 0]], axis=-1)
    e = jnp.where(a > 0, a, jnp.exp(jnp.minimum(a, 0.0)) - 1.0)
    out_ref[0] = x_ref[0] + e


def _sc_body(edges, esed_h, ghead, zagg, zden, aggout,
             esed_t, den_t, src0, src1, dst0, dst1,
             idx_g0, idx_g1, idx_d0, idx_d1, pay_a, pay_b, pay_c, pay_d,
             rows0, rows1, den_sh, agg_sh,
             sem_i0, sem_i1, sem_g0, sem_g1, sem_s0, sem_s1, sem_d):
    c = lax.axis_index("c")
    s = lax.axis_index("s")
    nb = s * NPT

    pltpu.sync_copy(zden, den_sh.at[pl.ds(s * (2 * NP2 // NS), 2 * NP2 // NS)])
    pltpu.sync_copy(esed_h.at[c], esed_t.at[pl.ds(0, N * 4)])
    esed_t[pl.ds(N * 4, 16)] = jnp.zeros((16,), jnp.float32)
    plsc.subcore_barrier()

    ebase = s * EPT
    isem = (sem_i0, sem_i1)
    srcb = (src0, src1)
    dstb = (dst0, dst1)

    def fire_idx(i, b):
        eb = ebase + i * CHUNK
        pltpu.async_copy(edges.at[0].at[pl.ds(eb, CHUNK)], srcb[b], isem[b])
        pltpu.async_copy(edges.at[1].at[pl.ds(eb, CHUNK)], dstb[b], isem[b])

    def wait_idx(b):
        pltpu.make_async_copy(edges.at[0].at[pl.ds(0, CHUNK)], srcb[b],
                              isem[b]).wait()
        pltpu.make_async_copy(edges.at[0].at[pl.ds(0, CHUNK)], dstb[b],
                              isem[b]).wait()

    def logits(sv, dv, h):
        es_v = plsc.load_gather(esed_t, [sv * 4 + h])
        ed_v = plsc.load_gather(esed_t, [dv * 4 + HEADS + h])
        e = es_v + ed_v
        return jnp.where(e > 0, e, 0.2 * e)

    p1buf = ((idx_d0, idx_g0, pay_a, pay_c), (idx_d1, idx_g1, pay_b, pay_d))

    def p1_compute(b):
        bd, bg, p0, p1x = p1buf[b]
        for j in range(CHUNK // 16):
            sl = pl.ds(j * 16, 16)
            sv = srcb[b][sl]
            dv = dstb[b][sl]
            bd[sl] = dv
            bg[sl] = dv + NP2
            p0[sl] = jnp.exp(logits(sv, dv, 0))
            p1x[sl] = jnp.exp(logits(sv, dv, 1))

    def p1_fire(b):
        bd, bg, p0, p1x = p1buf[b]
        pltpu.make_async_copy(p0, den_sh.at[bd], sem_d).start(add=True)
        pltpu.make_async_copy(p1x, den_sh.at[bg], sem_d).start(add=True)

    def p1_wait():
        pltpu.make_async_copy(pay_a, den_sh.at[idx_d0], sem_d).wait()
        pltpu.make_async_copy(pay_a, den_sh.at[idx_d0], sem_d).wait()

    fire_idx(0, 0)
    fire_idx(1, 1)
    wait_idx(0)
    p1_compute(0)
    p1_fire(0)
    fire_idx(2, 0)

    def p1_pair(k, _):
        wait_idx(1)
        p1_compute(1)

        @pl.when(2 * k + 3 < NCHT)
        def _():
            fire_idx(2 * k + 3, 1)

        p1_wait()
        p1_fire(1)
        wait_idx(0)
        p1_compute(0)

        @pl.when(2 * k + 4 < NCHT)
        def _():
            fire_idx(2 * k + 4, 0)

        p1_wait()
        p1_fire(0)
        return 0

    lax.fori_loop(0, (NCHT - 2) // 2, p1_pair, 0)
    wait_idx(1)
    p1_compute(1)
    p1_wait()
    p1_fire(1)
    p1_wait()
    plsc.subcore_barrier()

    pltpu.sync_copy(den_sh, den_t)

    def invd(q, _):
        sl = pl.ds(q * 16, 16)
        den_t[sl] = 1.0 / (den_t[sl] + 1e-16)
        return 0

    lax.fori_loop(0, 2 * NP2 // 16, invd, 0)

    for h in range(HEADS):
        pltpu.sync_copy(zagg, agg_sh.at[pl.ds(nb, NPT)])

        @pl.when(s == NS - 1)
        def _():
            pltpu.sync_copy(zagg.at[pl.ds(0, NTAIL)],
                            agg_sh.at[pl.ds(NPT * NS, NTAIL)])

        plsc.subcore_barrier()

        bufs = ((idx_g0, idx_d0, pay_a, rows0, sem_g0, sem_s0),
                (idx_g1, idx_d1, pay_b, rows1, sem_g1, sem_s1))

        def prep_fire_g(b):
            bg, _bd, _p, br, sg, _ss = bufs[b]
            for j in range(CHUNK // 16):
                sl = pl.ds(j * 16, 16)
                bg[sl] = (srcb[b][sl] + c * N) * 2 + h
            pltpu.async_copy(ghead.at[bg], br, sg)

        def alpha(b):
            _bg, bd, p, _br, _sg, _ss = bufs[b]
            for j in range(CHUNK // 16):
                sl = pl.ds(j * 16, 16)
                sv = srcb[b][sl]
                dv = dstb[b][sl]
                bd[sl] = jnp.minimum(dv, N - 1)
                dh = plsc.load_gather(den_t, [dv + h * NP2])
                av = jnp.exp(logits(sv, dv, h)) * dh
                p[sl] = jnp.where(dv < N, av, 0.0)

        def process(b):
            bg, bd, p, br, sg, ss = bufs[b]
            pltpu.make_async_copy(ghead.at[bg], br, sg).wait()

            def scale(q, _):
                for t in range(2):
                    jj = 2 * q + t
                    av = plsc.load_gather(p, [jnp.full((16,), jj, jnp.int32)])
                    for k in range(4):
                        br[jj, pl.ds(k * 16, 16)] = br[jj, pl.ds(k * 16, 16)] * av
                return 0

            lax.fori_loop(0, CHUNK // 2, scale, 0)
            pltpu.make_async_copy(br, agg_sh.at[bd], ss).start(add=True)

        def wait_s(b):
            _bg, bd, _p, br, _sg, ss = bufs[b]
            pltpu.make_async_copy(br, agg_sh.at[bd], ss).wait()

        fire_idx(0, 0)
        fire_idx(1, 1)
        wait_idx(0)
        prep_fire_g(0)
        wait_idx(1)
        prep_fire_g(1)

        def p2_pair(k, _):
            alpha(0)

            @pl.when(2 * k + 2 < NCHT)
            def _():
                fire_idx(2 * k + 2, 0)

            process(0)
            alpha(1)

            @pl.when(2 * k + 3 < NCHT)
            def _():
                fire_idx(2 * k + 3, 1)

            wait_s(0)

            @pl.when(2 * k + 2 < NCHT)
            def _():
                wait_idx(0)
                prep_fire_g(0)

            process(1)
            wait_s(1)

            @pl.when(2 * k + 3 < NCHT)
            def _():
                wait_idx(1)
                prep_fire_g(1)

            return 0

        lax.fori_loop(0, NCHT // 2, p2_pair, 0)
        plsc.subcore_barrier()

        pltpu.sync_copy(agg_sh.at[pl.ds(nb, NPT)],
                        aggout.at[h].at[pl.ds(c * N + nb, NPT)])

        @pl.when(s == NS - 1)
        def _():
            pltpu.sync_copy(agg_sh.at[pl.ds(NPT * NS, NTAIL)],
                            aggout.at[h].at[pl.ds(c * N + NPT * NS, NTAIL)])


_sc_kernel = functools.partial(
    pl.kernel,
    out_type=jax.ShapeDtypeStruct((HEADS, B * N, DH), jnp.float32),
    mesh=plsc.VectorSubcoreMesh(core_axis_name="c", subcore_axis_name="s",
                                num_cores=NC, num_subcores=NS),
    compiler_params=pltpu.CompilerParams(needs_layout_passes=False,
                                         use_tc_tiling_on_sc=False),
    scratch_types=[
        pltpu.VMEM((N * 2 * HEADS + 16,), jnp.float32),
        pltpu.VMEM((2 * NP2,), jnp.float32),
        pltpu.VMEM((CHUNK,), jnp.int32),
        pltpu.VMEM((CHUNK,), jnp.int32),
        pltpu.VMEM((CHUNK,), jnp.int32),
        pltpu.VMEM((CHUNK,), jnp.int32),
        pltpu.VMEM((CHUNK,), jnp.int32),
        pltpu.VMEM((CHUNK,), jnp.int32),
        pltpu.VMEM((CHUNK,), jnp.int32),
        pltpu.VMEM((CHUNK,), jnp.int32),
        pltpu.VMEM((CHUNK,), jnp.float32),
        pltpu.VMEM((CHUNK,), jnp.float32),
        pltpu.VMEM((CHUNK,), jnp.float32),
        pltpu.VMEM((CHUNK,), jnp.float32),
        pltpu.VMEM((CHUNK, DH), jnp.float32),
        pltpu.VMEM((CHUNK, DH), jnp.float32),
        pltpu.VMEM_SHARED((2 * NP2,), jnp.float32),
        pltpu.VMEM_SHARED((N, DH), jnp.float32),
        pltpu.SemaphoreType.DMA,
        pltpu.SemaphoreType.DMA,
        pltpu.SemaphoreType.DMA,
        pltpu.SemaphoreType.DMA,
        pltpu.SemaphoreType.DMA,
        pltpu.SemaphoreType.DMA,
        pltpu.SemaphoreType.DMA,
    ],
)(_sc_body)


def kernel(patch_features, edges, W1, b1, W2, b2, Wg, a_src, a_dst):
    z = jnp.zeros((DH,), jnp.float32)
    A = jnp.stack([
        jnp.concatenate([a_src[0], z]),
        jnp.concatenate([z, a_src[1]]),
        jnp.concatenate([a_dst[0], z]),
        jnp.concatenate([z, a_dst[1]]),
    ], axis=1)

    x, gflat, esed = pl.pallas_call(
        _tc_body,
        grid=(B, N // BLK),
        in_specs=[
            pl.BlockSpec((1, BLK, IN), lambda b, i: (b, i, 0)),
            pl.BlockSpec((IN, HID), lambda b, i: (0, 0)),
            pl.BlockSpec((1, HID), lambda b, i: (0, 0)),
            pl.BlockSpec((HID, OUT), lambda b, i: (0, 0)),
            pl.BlockSpec((1, OUT), lambda b, i: (0, 0)),
            pl.BlockSpec((OUT, F), lambda b, i: (0, 0)),
            pl.BlockSpec((F, 2 * HEADS), lambda b, i: (0, 0)),
        ],
        out_specs=[
            pl.BlockSpec((1, BLK, OUT), lambda b, i: (b, i, 0)),
            pl.BlockSpec((BLK, F), lambda b, i: (b * (N // BLK) + i, 0)),
            pl.BlockSpec((1, BLK, 2 * HEADS), lambda b, i: (b, i, 0)),
        ],
        out_shape=[
            jax.ShapeDtypeStruct((B, N, OUT), jnp.float32),
            jax.ShapeDtypeStruct((B * N, F), jnp.float32),
            jax.ShapeDtypeStruct((B, N, 2 * HEADS), jnp.float32),
        ],
    )(patch_features, W1, b1.reshape(1, HID), W2, b2.reshape(1, OUT), Wg, A)

    ghead = gflat.reshape(B * N * HEADS, DH)
    zagg = jnp.zeros((NPT, DH), jnp.float32)
    zden = jnp.zeros((2 * NP2 // NS,), jnp.float32)
    pad = jnp.stack([jnp.zeros((E_P - E,), jnp.int32),
                     jnp.full((E_P - E,), N, jnp.int32)])
    edges_p = jnp.concatenate([edges, pad], axis=1)
    agg = _sc_kernel(edges_p, esed.reshape(B, N * 2 * HEADS), ghead, zagg, zden)

    return pl.pallas_call(
        _tc2_body,
        grid=(B, N // BLK),
        in_specs=[
            pl.BlockSpec((1, BLK, OUT), lambda b, i: (b, i, 0)),
            pl.BlockSpec((1, 1, BLK, DH), lambda b, i: (0, b, i, 0)),
            pl.BlockSpec((1, 1, BLK, DH), lambda b, i: (1, b, i, 0)),
        ],
        out_specs=pl.BlockSpec((1, BLK, OUT), lambda b, i: (b, i, 0)),
        out_shape=jax.ShapeDtypeStruct((B, N, OUT), jnp.float32),
    )(x, agg.reshape(HEADS, B, N, DH), agg.reshape(HEADS, B, N, DH))
